# Initial kernel scaffold; baseline (speedup 1.0000x reference)
import jax
import jax.numpy as jnp
from jax.experimental import pallas as pl


def kernel(relative_position_bias_table, relative_position_index):
    raise NotImplementedError("write your pallas kernel here")


# speedup vs baseline: 4.6473x; 4.6473x over previous
import functools

import jax
import jax.numpy as jnp
from jax import lax
from jax.experimental import pallas as pl
from jax.experimental.pallas import tpu as pltpu
from jax.experimental.pallas import tpu_sc as plsc

H, W = 32, 32
N = H * W + 1
NUM_HEADS = 16
NUM_REL = (2 * H - 1) * (2 * W - 1) + 3
TABLE_FLAT = NUM_REL * NUM_HEADS

NC, NS, L = 2, 16, 16
NW = NC * NS
ROWS_PER_TILE = N // NW
R = 8
FULL_VPR = N // L
TAIL_POS = N - L


def _sc_gather(table_flat, idx):
    mesh = plsc.VectorSubcoreMesh(
        core_axis_name="c", subcore_axis_name="s", num_cores=NC, num_subcores=NS
    )

    @functools.partial(
        pl.kernel,
        out_type=jax.ShapeDtypeStruct((NUM_HEADS, N, N), jnp.float32),
        mesh=mesh,
        compiler_params=pltpu.CompilerParams(
            use_tc_tiling_on_sc=False, needs_layout_passes=False
        ),
        scratch_types=[
            pltpu.VMEM((TABLE_FLAT,), jnp.float32),
            pltpu.VMEM((R, N), jnp.int32),
            pltpu.VMEM((R, N), jnp.float32),
        ],
    )
    def k(table_hbm, idx_hbm, out_hbm, table_v, idx_v, out_v):
        wid = lax.axis_index("s") * NC + lax.axis_index("c")
        pltpu.sync_copy(table_hbm, table_v)
        tail_cols = lax.iota(jnp.int32, L) + TAIL_POS

        def process_chunk(r0):
            pltpu.sync_copy(idx_hbm.at[pl.ds(r0, R), :], idx_v)
            for h in range(NUM_HEADS):
                def body(t, _, h=h):
                    r = t // FULL_VPR
                    pos = pl.multiple_of((t % FULL_VPR) * L, L)
                    iv = idx_v[r, pl.ds(pos, L)]
                    out_v[r, pl.ds(pos, L)] = plsc.load_gather(
                        table_v, [iv * NUM_HEADS + h]
                    )
                    return 0

                lax.fori_loop(0, R * FULL_VPR, body, 0)

                def tail(r, _, h=h):
                    rvec = jnp.full((L,), r, jnp.int32)
                    iv = plsc.load_gather(idx_v, [rvec, tail_cols])
                    vals = plsc.load_gather(table_v, [iv * NUM_HEADS + h])
                    plsc.store_scatter(out_v, [rvec, tail_cols], vals)
                    return 0

                lax.fori_loop(0, R, tail, 0)
                pltpu.sync_copy(out_v, out_hbm.at[h, pl.ds(r0, R), :])

        base = wid * ROWS_PER_TILE
        for c in range(ROWS_PER_TILE // R):
            process_chunk(base + c * R)

        @pl.when(wid == NW - 1)
        def _():
            process_chunk(N - R)

    return k(table_flat, idx)


def kernel(relative_position_bias_table, relative_position_index):
    table_flat = relative_position_bias_table.reshape(-1)
    idx = relative_position_index.astype(jnp.int32)
    out = _sc_gather(table_flat, idx)
    return out.reshape(1, NUM_HEADS, N, N)
